# baseline (speedup 1.0000x reference)
import functools

import jax
import jax.numpy as jnp
from jax import lax
from jax.experimental import pallas as pl
from jax.experimental.pallas import tpu as pltpu
from jax.experimental.pallas import tpu_sc as plsc

NUM_ACT = 96
SIZE = 256
B, H, W = 8, 224, 224
NSLICES = B * NUM_ACT
NWORKERS = 32
SLICES_PER_W = NSLICES // NWORKERS
RBLK = H // 2
NVROW = W // 16

_mesh = plsc.VectorSubcoreMesh(
    core_axis_name="c", subcore_axis_name="s", num_cores=2, num_subcores=16
)


@functools.partial(
    pl.kernel,
    out_type=jax.ShapeDtypeStruct((NSLICES, H, W), jnp.float32),
    mesh=_mesh,
    compiler_params=pltpu.CompilerParams(needs_layout_passes=False),
    scratch_types=[
        pltpu.VMEM((SIZE + 128,), jnp.float32),
        pltpu.VMEM((SIZE,), jnp.float32),
        pltpu.VMEM((128,), jnp.float32),
        pltpu.VMEM((64,), jnp.float32),
        pltpu.VMEM((RBLK, W), jnp.float32),
        pltpu.VMEM((RBLK, W), jnp.float32),
        pltpu.VMEM((RBLK, W), jnp.float32),
        pltpu.VMEM((RBLK, W), jnp.float32),
        pltpu.SemaphoreType.DMA,
        pltpu.SemaphoreType.DMA,
        pltpu.SemaphoreType.DMA,
        pltpu.SemaphoreType.DMA,
    ],
)
def _spline_sc(x_hbm, coef_hbm, scal_hbm, par_hbm, out_hbm,
               tab, tabd, scal_v, par_v, xb0, xb1, ob0, ob1,
               si0, si1, so0, so1):
    cid = lax.axis_index("c")
    sid = lax.axis_index("s")
    w = sid * 2 + cid
    sl0 = w * SLICES_PER_W

    xb = (xb0, xb1)
    ob = (ob0, ob1)
    si = (si0, si1)
    so = (so0, so1)

    pltpu.sync_copy(scal_hbm, scal_v.at[pl.ds(0, NUM_ACT)])
    pltpu.sync_copy(par_hbm, par_v)
    g0 = par_v[pl.ds(0, 16)]
    tmax = par_v[pl.ds(16, 16)]
    invh = par_v[pl.ds(32, 16)]
    halfr = par_v[pl.ds(48, 16)]
    zero = g0 - g0

    pltpu.async_copy(x_hbm.at[sl0, pl.ds(0, RBLK)], xb0, si0)

    def outer(k, _):
        sl = sl0 + k
        for b in range(2):
            if b == 0:
                pltpu.async_copy(x_hbm.at[sl, pl.ds(RBLK, RBLK)], xb1, si1)
            else:
                @pl.when(k < SLICES_PER_W - 1)
                def _prefetch():
                    pltpu.async_copy(x_hbm.at[sl + 1, pl.ds(0, RBLK)], xb0, si0)

            if b == 0:
                ch = lax.rem(sl, NUM_ACT)
                pltpu.sync_copy(
                    coef_hbm.at[pl.ds(pl.multiple_of(ch * SIZE, SIZE), SIZE)],
                    tab.at[pl.ds(0, SIZE)],
                )
                chv = jnp.full((16,), ch, jnp.int32)
                sv = plsc.load_gather(scal_v, [chv])
                kv = halfr * sv
                iota = lax.iota(jnp.int32, 16)

                @plsc.parallel_loop(0, SIZE // 16)
                def _tscale(i):
                    off = pl.multiple_of(i * 16, 16)
                    tab[pl.ds(off, 16)] = tab[pl.ds(off, 16)] * sv + kv

                @plsc.parallel_loop(0, SIZE // 16)
                def _tdelta(i):
                    off = pl.multiple_of(i * 16, 16)
                    nxt = plsc.load_gather(tab, [iota + (off + 1)])
                    tabd[pl.ds(off, 16)] = nxt - tab[pl.ds(off, 16)]

            pltpu.make_async_copy(
                x_hbm.at[0, pl.ds(0, RBLK)], xb[b], si[b]
            ).wait()

            @pl.when(k >= 1)
            def _drain_prev():
                pltpu.make_async_copy(
                    x_hbm.at[0, pl.ds(0, RBLK)], ob[b], so[b]
                ).wait()

            xbuf = xb[b]
            obuf = ob[b]

            @plsc.parallel_loop(0, RBLK, unroll=2)
            def _row(r):
                for i in range(NVROW):
                    off = i * 16
                    xv = xbuf[r, pl.ds(off, 16)]
                    t = jnp.minimum(jnp.maximum(xv * invh - g0, zero), tmax)
                    li = t.astype(jnp.int32)
                    fr = t - li.astype(jnp.float32)
                    cl = plsc.load_gather(tab, [li])
                    d = plsc.load_gather(tabd, [li])
                    obuf[r, pl.ds(off, 16)] = cl + fr * d

            pltpu.async_copy(obuf, out_hbm.at[sl, pl.ds(b * RBLK, RBLK)], so[b])
        return 0

    lax.fori_loop(0, SLICES_PER_W, outer, 0)

    for b in range(2):
        pltpu.make_async_copy(x_hbm.at[0, pl.ds(0, RBLK)], ob[b], so[b]).wait()


def kernel(x, coefficients_vect, scaling_coeffs_vect, grid):
    x3 = x.reshape(NSLICES, H, W)
    scal = scaling_coeffs_vect.reshape(-1).astype(jnp.float32)
    gmin = grid[0]
    gmax = grid[-1]
    invh = (SIZE - 1) / (gmax - gmin)
    halfr = jnp.where(SIZE % 2 == 0, (gmax - gmin) / 2.0, 0.0)
    tmax = jnp.float32(SIZE - 1) - jnp.float32(SIZE - 1) * jnp.float32(2.0) ** -23
    par = jnp.concatenate([
        jnp.full((16,), gmin * invh, jnp.float32),
        jnp.full((16,), tmax, jnp.float32),
        jnp.full((16,), invh, jnp.float32),
        jnp.full((16,), halfr, jnp.float32),
    ])
    out = _spline_sc(x3, coefficients_vect.astype(jnp.float32), scal, par)
    return out.reshape(x.shape)


# speedup vs baseline: 1.1064x; 1.1064x over previous
import functools

import jax
import jax.numpy as jnp
from jax import lax
from jax.experimental import pallas as pl
from jax.experimental.pallas import tpu as pltpu
from jax.experimental.pallas import tpu_sc as plsc

NUM_ACT = 96
SIZE = 256
B, H, W = 8, 224, 224
NSLICES = B * NUM_ACT
NWORKERS = 32
SLICES_PER_W = NSLICES // NWORKERS
RBLK = H // 2
NVROW = W // 16

_mesh = plsc.VectorSubcoreMesh(
    core_axis_name="c", subcore_axis_name="s", num_cores=2, num_subcores=16
)


@functools.partial(
    pl.kernel,
    out_type=jax.ShapeDtypeStruct((NSLICES, H, W), jnp.float32),
    mesh=_mesh,
    compiler_params=pltpu.CompilerParams(needs_layout_passes=False),
    scratch_types=[
        pltpu.VMEM((SIZE + 128,), jnp.float32),
        pltpu.VMEM((SIZE,), jnp.float32),
        pltpu.VMEM((SIZE,), jnp.float32),
        pltpu.VMEM((SIZE,), jnp.float32),
        pltpu.VMEM((128,), jnp.float32),
        pltpu.VMEM((64,), jnp.float32),
        pltpu.VMEM((RBLK, W), jnp.float32),
        pltpu.VMEM((RBLK, W), jnp.float32),
        pltpu.VMEM((RBLK, W), jnp.float32),
        pltpu.VMEM((RBLK, W), jnp.float32),
        pltpu.SemaphoreType.DMA,
        pltpu.SemaphoreType.DMA,
        pltpu.SemaphoreType.DMA,
        pltpu.SemaphoreType.DMA,
        pltpu.SemaphoreType.DMA,
        pltpu.SemaphoreType.DMA,
    ],
)
def _spline_sc(x_hbm, coef_hbm, scal_hbm, par_hbm, out_hbm,
               tab, tabd, traw0, traw1, scal_v, par_v, xb0, xb1, ob0, ob1,
               si0, si1, so0, so1, st0, st1):
    cid = lax.axis_index("c")
    sid = lax.axis_index("s")
    w = sid * 2 + cid
    sl0 = w * SLICES_PER_W

    xb = (xb0, xb1)
    ob = (ob0, ob1)
    si = (si0, si1)
    so = (so0, so1)
    traw = (traw0, traw1)
    st = (st0, st1)

    pltpu.sync_copy(scal_hbm, scal_v.at[pl.ds(0, NUM_ACT)])
    pltpu.sync_copy(par_hbm, par_v)
    g0 = par_v[pl.ds(0, 16)]
    tmax = par_v[pl.ds(16, 16)]
    invh = par_v[pl.ds(32, 16)]
    halfr = par_v[pl.ds(48, 16)]
    zero = g0 - g0

    def raw_table_copy(sl, p):
        ch = lax.rem(sl, NUM_ACT)
        pltpu.async_copy(
            coef_hbm.at[pl.ds(pl.multiple_of(ch * SIZE, SIZE), SIZE)],
            traw[p], st[p],
        )

    pltpu.async_copy(x_hbm.at[sl0, pl.ds(0, RBLK)], xb0, si0)
    raw_table_copy(sl0, 0)

    iota = lax.iota(jnp.int32, 16)

    def outer(o, _):
        for p in range(2):
            k = o * 2 + p
            sl = sl0 + k
            for b in range(2):
                if b == 0:
                    pltpu.async_copy(x_hbm.at[sl, pl.ds(RBLK, RBLK)], xb1, si1)
                else:
                    @pl.when(k < SLICES_PER_W - 1)
                    def _prefetch():
                        pltpu.async_copy(
                            x_hbm.at[sl + 1, pl.ds(0, RBLK)], xb0, si0
                        )

                if b == 0:
                    pltpu.make_async_copy(
                        coef_hbm.at[pl.ds(0, SIZE)], traw[p], st[p]
                    ).wait()
                    ch = lax.rem(sl, NUM_ACT)
                    chv = jnp.full((16,), ch, jnp.int32)
                    sv = plsc.load_gather(scal_v, [chv])
                    kv = halfr * sv
                    rawp = traw[p]

                    @plsc.parallel_loop(0, SIZE // 16)
                    def _tscale(i):
                        off = pl.multiple_of(i * 16, 16)
                        tab[pl.ds(off, 16)] = rawp[pl.ds(off, 16)] * sv + kv

                    @plsc.parallel_loop(0, SIZE // 16)
                    def _tdelta(i):
                        off = pl.multiple_of(i * 16, 16)
                        nxt = plsc.load_gather(tab, [iota + (off + 1)])
                        tabd[pl.ds(off, 16)] = nxt - tab[pl.ds(off, 16)]

                    if p == 0:
                        raw_table_copy(sl + 1, 1)
                    else:
                        @pl.when(o < SLICES_PER_W // 2 - 1)
                        def _next_table():
                            raw_table_copy(sl + 1, 0)

                pltpu.make_async_copy(
                    x_hbm.at[0, pl.ds(0, RBLK)], xb[b], si[b]
                ).wait()

                @pl.when(k >= 1)
                def _drain_prev():
                    pltpu.make_async_copy(
                        x_hbm.at[0, pl.ds(0, RBLK)], ob[b], so[b]
                    ).wait()

                xbuf = xb[b]
                obuf = ob[b]

                @plsc.parallel_loop(0, RBLK)
                def _row(r):
                    for i in range(NVROW):
                        off = i * 16
                        xv = xbuf[r, pl.ds(off, 16)]
                        t = jnp.minimum(jnp.maximum(xv * invh - g0, zero), tmax)
                        li = t.astype(jnp.int32)
                        fr = t - li.astype(jnp.float32)
                        cl = plsc.load_gather(tab, [li])
                        d = plsc.load_gather(tabd, [li])
                        obuf[r, pl.ds(off, 16)] = cl + fr * d

                pltpu.async_copy(
                    obuf, out_hbm.at[sl, pl.ds(b * RBLK, RBLK)], so[b]
                )
        return 0

    lax.fori_loop(0, SLICES_PER_W // 2, outer, 0)

    for b in range(2):
        pltpu.make_async_copy(x_hbm.at[0, pl.ds(0, RBLK)], ob[b], so[b]).wait()


def kernel(x, coefficients_vect, scaling_coeffs_vect, grid):
    x3 = x.reshape(NSLICES, H, W)
    scal = scaling_coeffs_vect.reshape(-1).astype(jnp.float32)
    gmin = grid[0]
    gmax = grid[-1]
    invh = (SIZE - 1) / (gmax - gmin)
    halfr = jnp.where(SIZE % 2 == 0, (gmax - gmin) / 2.0, 0.0)
    tmax = jnp.float32(SIZE - 1) - jnp.float32(SIZE - 1) * jnp.float32(2.0) ** -23
    par = jnp.concatenate([
        jnp.full((16,), gmin * invh, jnp.float32),
        jnp.full((16,), tmax, jnp.float32),
        jnp.full((16,), invh, jnp.float32),
        jnp.full((16,), halfr, jnp.float32),
    ])
    out = _spline_sc(x3, coefficients_vect.astype(jnp.float32), scal, par)
    return out.reshape(x.shape)


# speedup vs baseline: 1.1763x; 1.0632x over previous
import functools

import jax
import jax.numpy as jnp
from jax import lax
from jax.experimental import pallas as pl
from jax.experimental.pallas import tpu as pltpu
from jax.experimental.pallas import tpu_sc as plsc

NUM_ACT = 96
SIZE = 256
B, H, W = 8, 224, 224
NSLICES = B * NUM_ACT
NWORKERS = 32
SLICES_PER_W = NSLICES // NWORKERS
RBLK = H // 2
NVROW = W // 16

_mesh = plsc.VectorSubcoreMesh(
    core_axis_name="c", subcore_axis_name="s", num_cores=2, num_subcores=16
)


@functools.partial(
    pl.kernel,
    out_type=jax.ShapeDtypeStruct((NSLICES, H, W), jnp.float32),
    mesh=_mesh,
    compiler_params=pltpu.CompilerParams(needs_layout_passes=False),
    scratch_types=[
        pltpu.VMEM((SIZE + 128,), jnp.float32),
        pltpu.VMEM((SIZE,), jnp.float32),
        pltpu.VMEM((SIZE,), jnp.float32),
        pltpu.VMEM((SIZE,), jnp.float32),
        pltpu.VMEM((128,), jnp.float32),
        pltpu.VMEM((64,), jnp.float32),
        pltpu.VMEM((RBLK, W), jnp.float32),
        pltpu.VMEM((RBLK, W), jnp.float32),
        pltpu.VMEM((RBLK, W), jnp.float32),
        pltpu.VMEM((RBLK, W), jnp.float32),
        pltpu.SemaphoreType.DMA,
        pltpu.SemaphoreType.DMA,
        pltpu.SemaphoreType.DMA,
        pltpu.SemaphoreType.DMA,
        pltpu.SemaphoreType.DMA,
        pltpu.SemaphoreType.DMA,
    ],
)
def _spline_sc(x_hbm, coef_hbm, scal_hbm, par_hbm, out_hbm,
               tab, tabd, traw0, traw1, scal_v, par_v, xb0, xb1, ob0, ob1,
               si0, si1, so0, so1, st0, st1):
    cid = lax.axis_index("c")
    sid = lax.axis_index("s")
    w = sid * 2 + cid
    sl0 = w * SLICES_PER_W

    xb = (xb0, xb1)
    ob = (ob0, ob1)
    si = (si0, si1)
    so = (so0, so1)
    traw = (traw0, traw1)
    st = (st0, st1)

    pltpu.sync_copy(scal_hbm, scal_v.at[pl.ds(0, NUM_ACT)])
    pltpu.sync_copy(par_hbm, par_v)
    g0 = par_v[pl.ds(0, 16)]
    tmax = par_v[pl.ds(16, 16)]
    invh = par_v[pl.ds(32, 16)]
    halfr = par_v[pl.ds(48, 16)]
    zero = g0 - g0

    def raw_table_copy(sl, p):
        ch = lax.rem(sl, NUM_ACT)
        pltpu.async_copy(
            coef_hbm.at[pl.ds(pl.multiple_of(ch * SIZE, SIZE), SIZE)],
            traw[p], st[p],
        )

    pltpu.async_copy(x_hbm.at[sl0, pl.ds(0, RBLK)], xb0, si0)
    raw_table_copy(sl0, 0)

    iota = lax.iota(jnp.int32, 16)

    def outer(o, _):
        for p in range(2):
            k = o * 2 + p
            sl = sl0 + k
            for b in range(2):
                if b == 0:
                    pltpu.async_copy(x_hbm.at[sl, pl.ds(RBLK, RBLK)], xb1, si1)
                else:
                    @pl.when(k < SLICES_PER_W - 1)
                    def _prefetch():
                        pltpu.async_copy(
                            x_hbm.at[sl + 1, pl.ds(0, RBLK)], xb0, si0
                        )

                if b == 0:
                    pltpu.make_async_copy(
                        coef_hbm.at[pl.ds(0, SIZE)], traw[p], st[p]
                    ).wait()
                    ch = lax.rem(sl, NUM_ACT)
                    chv = jnp.full((16,), ch, jnp.int32)
                    sv = plsc.load_gather(scal_v, [chv])
                    kv = halfr * sv
                    rawp = traw[p]

                    @plsc.parallel_loop(0, SIZE // 16)
                    def _tscale(i):
                        off = pl.multiple_of(i * 16, 16)
                        tab[pl.ds(off, 16)] = rawp[pl.ds(off, 16)] * sv + kv

                    @plsc.parallel_loop(0, SIZE // 16)
                    def _tdelta(i):
                        off = pl.multiple_of(i * 16, 16)
                        nxt = plsc.load_gather(tab, [iota + (off + 1)])
                        tabd[pl.ds(off, 16)] = nxt - tab[pl.ds(off, 16)]

                    @plsc.parallel_loop(0, SIZE // 16)
                    def _trebase(i):
                        off = pl.multiple_of(i * 16, 16)
                        lf = (iota + off).astype(jnp.float32)
                        tab[pl.ds(off, 16)] = (
                            tab[pl.ds(off, 16)] - lf * tabd[pl.ds(off, 16)]
                        )

                    if p == 0:
                        raw_table_copy(sl + 1, 1)
                    else:
                        @pl.when(o < SLICES_PER_W // 2 - 1)
                        def _next_table():
                            raw_table_copy(sl + 1, 0)

                pltpu.make_async_copy(
                    x_hbm.at[0, pl.ds(0, RBLK)], xb[b], si[b]
                ).wait()

                @pl.when(k >= 1)
                def _drain_prev():
                    pltpu.make_async_copy(
                        x_hbm.at[0, pl.ds(0, RBLK)], ob[b], so[b]
                    ).wait()

                xbuf = xb[b]
                obuf = ob[b]

                @plsc.parallel_loop(0, RBLK)
                def _row(r):
                    for i in range(NVROW):
                        off = i * 16
                        xv = xbuf[r, pl.ds(off, 16)]
                        t = jnp.minimum(jnp.maximum(xv * invh - g0, zero), tmax)
                        li = t.astype(jnp.int32)
                        a = plsc.load_gather(tab, [li])
                        d = plsc.load_gather(tabd, [li])
                        obuf[r, pl.ds(off, 16)] = a + t * d

                pltpu.async_copy(
                    obuf, out_hbm.at[sl, pl.ds(b * RBLK, RBLK)], so[b]
                )
        return 0

    lax.fori_loop(0, SLICES_PER_W // 2, outer, 0)

    for b in range(2):
        pltpu.make_async_copy(x_hbm.at[0, pl.ds(0, RBLK)], ob[b], so[b]).wait()


def kernel(x, coefficients_vect, scaling_coeffs_vect, grid):
    x3 = x.reshape(NSLICES, H, W)
    scal = scaling_coeffs_vect.reshape(-1).astype(jnp.float32)
    gmin = grid[0]
    gmax = grid[-1]
    invh = (SIZE - 1) / (gmax - gmin)
    halfr = jnp.where(SIZE % 2 == 0, (gmax - gmin) / 2.0, 0.0)
    tmax = jnp.float32(SIZE - 1) - jnp.float32(SIZE - 1) * jnp.float32(2.0) ** -23
    par = jnp.concatenate([
        jnp.full((16,), gmin * invh, jnp.float32),
        jnp.full((16,), tmax, jnp.float32),
        jnp.full((16,), invh, jnp.float32),
        jnp.full((16,), halfr, jnp.float32),
    ])
    out = _spline_sc(x3, coefficients_vect.astype(jnp.float32), scal, par)
    return out.reshape(x.shape)
